# baseline (speedup 1.0000x reference)
import jax
import jax.numpy as jnp
from jax import lax
from jax.experimental import pallas as pl
from jax.experimental.pallas import tpu as pltpu
from jax.experimental.pallas import tpu_sc as plsc

C = 16
D = 32
V = 1000000
N = 16384
I = 4096

NC = 2
NS = 16
NW = NC * NS
ROWS_PER_W = N // NW
KCH = ROWS_PER_W // 128
STRIPE = I // NS

OWN = 32000
CHW = 1280
TAIL0 = 999936
MAXO = 768
MAXC = 96
SENT = 1 << 29
OUTROWS = N + NW * MAXC


def _iota16():
    return lax.iota(jnp.int32, 16)


def _full16(v):
    return jnp.full((16,), v, jnp.int32)


def _splat_last(cs):
    dn = lax.GatherDimensionNumbers(offset_dims=(), collapsed_slice_dims=(0,),
                                    start_index_map=(0,))
    return lax.gather(cs, _full16(15).reshape(16, 1), dn, (1,),
                      mode=lax.GatherScatterMode.PROMISE_IN_BOUNDS)


def _inrange(av, c0, c1):
    t = lax.shift_right_logical(av - c0, 31)
    u = lax.shift_right_logical((c1 - 1) - av, 31)
    return (1 - t) * (1 - u)


def _gather_sc(tT_hbm, ids_hbm, tail_hbm, out_hbm,
               ids_v, slab0, slab1, tail_v, oid, onn, cid, cnn, cnn2, stg,
               sem0, sem1):
    wid = lax.axis_index("s") * NC + lax.axis_index("c")
    lo = wid * OWN
    hi = lo + OWN

    pltpu.sync_copy(tail_hbm, tail_v)

    def _c0(t):
        return pl.multiple_of(jnp.minimum(lo + t * CHW, TAIL0 - CHW), 128)

    NCHUNKS = OWN // CHW
    slabs = [slab0, slab1]
    sems = [sem0, sem1]
    for b in range(2):
        pltpu.async_copy(tT_hbm.at[:, pl.ds(_c0(b), CHW)], slabs[b], sems[b])

    def _pf(i, c):
        oid[i, :] = _full16(SENT)
        return c
    lax.fori_loop(0, MAXO // 16 + 1, _pf, 0)

    cnt0 = _full16(0)
    for p in range(2):
        pltpu.sync_copy(ids_hbm.at[pl.ds(p * (N // 2), N // 2)], ids_v)

        def _scan(i, cnt, p=p):
            av = ids_v[pl.ds(i * 16, 16)]
            mi = _inrange(av, lo, hi)
            cs = plsc.cumsum(mi)
            pos = MAXO + mi * (cnt + cs - 1 - MAXO)
            ph = lax.shift_right_logical(pos, 4)
            plo = pos - ph * 16
            plsc.store_scatter(oid, [ph, plo], av)
            plsc.store_scatter(onn, [ph, plo],
                               _iota16() + (i * 16 + p * (N // 2)))
            return jnp.minimum(cnt + _splat_last(cs), MAXO - 16)
        cnt0 = lax.fori_loop(0, N // 32, _scan, cnt0)

    dump = N + wid * MAXC
    def _emit_block(c0, c1, gather_fn):
        for k in range(MAXC // 16 + 1):
            cnn[k, :] = _iota16() + (dump + k * 16)
            cid[k, :] = _full16(0)

        def _filt(i, cc):
            av = oid[i, :]
            mi = _inrange(av, c0, c1)
            cs = plsc.cumsum(mi)
            pos = MAXC + mi * (cc + cs - 1 - MAXC)
            ph = lax.shift_right_logical(pos, 4)
            plo = pos - ph * 16
            plsc.store_scatter(cid, [ph, plo], (av - c0) * mi)
            plsc.store_scatter(cnn, [ph, plo], onn[i, :])
            return jnp.minimum(cc + _splat_last(cs), MAXC - 16)
        lax.fori_loop(0, MAXO // 16, _filt, _full16(0))

        def _grp(g, c):
            cols = cid[g, :]
            ridx = _iota16() + g * 16
            for d in range(D):
                plsc.store_scatter(stg, [ridx, _full16(d)],
                                   gather_fn(d, cols))
            return c
        lax.fori_loop(0, MAXC // 16, _grp, 0)

        for k in range(MAXC // 16):
            cnn2[0, pl.ds(k * 16, 16)] = cnn[k, :]
        pltpu.sync_copy(stg, out_hbm.at[cnn2.at[0]])

    def _ring(o, carry):
        for b in range(2):
            t = 2 * o + b
            pltpu.make_async_copy(tT_hbm.at[:, pl.ds(0, CHW)],
                                  slabs[b], sems[b]).wait()
            c0 = _c0(t)
            _emit_block(c0, c0 + CHW,
                        lambda d, cols, slab=slabs[b]:
                        plsc.load_gather(slab, [_full16(d), cols]))
            pltpu.async_copy(tT_hbm.at[:, pl.ds(_c0(t + 2), CHW)],
                             slabs[b], sems[b])
        return carry
    lax.fori_loop(0, (NCHUNKS - 1) // 2, _ring, 0)

    pltpu.make_async_copy(tT_hbm.at[:, pl.ds(0, CHW)],
                          slabs[0], sems[0]).wait()
    cL = _c0(NCHUNKS - 1)
    _emit_block(cL, cL + CHW,
                lambda d, cols: plsc.load_gather(slab0, [_full16(d), cols]))
    pltpu.make_async_copy(tT_hbm.at[:, pl.ds(0, CHW)],
                          slabs[1], sems[1]).wait()

    def _tail_gather(d, cols):
        fl = cols * D + d
        fh = lax.shift_right_logical(fl, 7)
        return plsc.load_gather(tail_v, [fh, fl - fh * 128])
    _emit_block(TAIL0, V, _tail_gather)


def _ll_tc(rows_ref, mu_ref, anno_ref, conf_ref, out_ref):
    rows = rows_ref[:, :D]
    e_mu = jnp.exp(mu_ref[...])
    ee = jnp.exp(e_mu)
    er = jnp.exp(rows)
    s = lax.dot_general(er, ee, (((1,), (1,)), ((), ())),
                        preferred_element_type=jnp.float32)
    anno = anno_ref[...]
    onehot = (anno == lax.broadcasted_iota(jnp.int32, rows.shape, 1)
              ).astype(jnp.float32)
    r_an = jnp.sum(rows * onehot, axis=1, keepdims=True)
    e_an = lax.dot_general(onehot, e_mu, (((1,), (1,)), ((), ())),
                           preferred_element_type=jnp.float32)
    out_ref[...] = (e_an + r_an - jnp.log(s)) * conf_ref[...]


def _scatter_sc(ll_hbm, items_hbm, out_hbm, idx_v, ll_v, zbuf, acc_sh, sem):
    del sem
    cid_ = lax.axis_index("c")
    sid = lax.axis_index("s")
    wid = sid * NC + cid_
    base = wid * ROWS_PER_W
    pltpu.sync_copy(items_hbm.at[wid], idx_v)
    pltpu.sync_copy(ll_hbm.at[pl.ds(base, ROWS_PER_W)], ll_v)

    def _zero_row(j, carry):
        zbuf[j, :] = jnp.zeros((C,), jnp.float32)
        return carry
    lax.fori_loop(0, STRIPE, _zero_row, 0)
    pltpu.sync_copy(zbuf, acc_sh.at[pl.ds(sid * STRIPE, STRIPE)])
    plsc.subcore_barrier()
    for j in range(KCH):
        pltpu.sync_copy(ll_v.at[pl.ds(j * 128, 128)],
                        acc_sh.at[idx_v.at[j]], add=True)
    plsc.subcore_barrier()
    pltpu.sync_copy(acc_sh.at[pl.ds(sid * STRIPE, STRIPE)], zbuf)
    pltpu.sync_copy(zbuf, out_hbm.at[cid_, pl.ds(sid * STRIPE, STRIPE)])


def _combine_tc(parts_ref, out_ref):
    out_ref[...] = (parts_ref[0] + parts_ref[1]).T


def kernel(mu, random_effects, anno, items, annotators, confidences):
    mesh = plsc.VectorSubcoreMesh(core_axis_name="c", subcore_axis_name="s")

    gather = pl.kernel(
        _gather_sc, mesh=mesh,
        compiler_params=pltpu.CompilerParams(needs_layout_passes=False),
        out_type=jax.ShapeDtypeStruct((OUTROWS, 128), jnp.float32),
        scratch_types=[
            pltpu.VMEM((N // 2,), jnp.int32),
            pltpu.VMEM((D, CHW), jnp.float32),
            pltpu.VMEM((D, CHW), jnp.float32),
            pltpu.VMEM(((V - TAIL0) * D // 128, 128), jnp.float32),
            pltpu.VMEM((MAXO // 16 + 1, 16), jnp.int32),
            pltpu.VMEM((MAXO // 16 + 1, 16), jnp.int32),
            pltpu.VMEM((MAXC // 16 + 1, 16), jnp.int32),
            pltpu.VMEM((MAXC // 16 + 1, 16), jnp.int32),
            pltpu.VMEM((1, MAXC), jnp.int32),
            pltpu.VMEM((MAXC, 128), jnp.float32),
            pltpu.SemaphoreType.DMA,
            pltpu.SemaphoreType.DMA,
        ],
    )
    rows4 = gather(random_effects.T, annotators.astype(jnp.int32),
                   lax.slice(random_effects, (TAIL0, 0), (V, D))
                   .reshape((V - TAIL0) * D // 128, 128))

    grid = 8
    blk = N // grid
    ll = pl.pallas_call(
        _ll_tc,
        grid=(grid,),
        in_specs=[
            pl.BlockSpec((blk, 128), lambda i: (i, 0)),
            pl.BlockSpec((C, D), lambda i: (0, 0)),
            pl.BlockSpec((blk, 1), lambda i: (i, 0)),
            pl.BlockSpec((blk, 1), lambda i: (i, 0)),
        ],
        out_specs=pl.BlockSpec((blk, C), lambda i: (i, 0)),
        out_shape=jax.ShapeDtypeStruct((N, C), jnp.float32),
    )(rows4, mu, anno.astype(jnp.int32).reshape(N, 1),
      confidences.reshape(N, 1))

    scatter = pl.kernel(
        _scatter_sc, mesh=mesh,
        compiler_params=pltpu.CompilerParams(use_tc_tiling_on_sc=False),
        out_type=jax.ShapeDtypeStruct((NC, I, C), jnp.float32),
        scratch_types=[
            pltpu.VMEM((KCH, 128), jnp.int32),
            pltpu.VMEM((ROWS_PER_W, C), jnp.float32),
            pltpu.VMEM((STRIPE, C), jnp.float32),
            pltpu.VMEM_SHARED((I, C), jnp.float32),
            pltpu.SemaphoreType.DMA,
        ],
    )
    parts = scatter(ll, items.astype(jnp.int32).reshape(NW, KCH, 128))

    return pl.pallas_call(
        _combine_tc,
        out_shape=jax.ShapeDtypeStruct((C, I), jnp.float32),
    )(parts)


# speedup vs baseline: 1.1205x; 1.1205x over previous
import jax
import jax.numpy as jnp
from jax import lax
from jax.experimental import pallas as pl
from jax.experimental.pallas import tpu as pltpu
from jax.experimental.pallas import tpu_sc as plsc

C = 16
D = 32
V = 1000000
N = 16384
I = 4096

NC = 2
NS = 16
NW = NC * NS
ROWS_PER_W = N // NW
KCH = ROWS_PER_W // 128
STRIPE = I // NS

OWN = 31744
CHW = 1024
TAIL0 = 999936
MAXO = 768
MAXC = 48
SENT = 1 << 29
OUTROWS = N + NW * MAXC


def _iota16():
    return lax.iota(jnp.int32, 16)


def _full16(v):
    return jnp.full((16,), v, jnp.int32)


def _splat_last(cs):
    dn = lax.GatherDimensionNumbers(offset_dims=(), collapsed_slice_dims=(0,),
                                    start_index_map=(0,))
    return lax.gather(cs, _full16(15).reshape(16, 1), dn, (1,),
                      mode=lax.GatherScatterMode.PROMISE_IN_BOUNDS)


def _inrange(av, c0, c1):
    t = lax.shift_right_logical(av - c0, 31)
    u = lax.shift_right_logical((c1 - 1) - av, 31)
    return (1 - t) * (1 - u)


def _gather_sc(tT_hbm, ids_hbm, tail_hbm, out_hbm,
               ids_v, slab0, slab1, tail_v, oid, onn, cid, cnn, cnn2, stg,
               sem0, sem1):
    wid = lax.axis_index("s") * NC + lax.axis_index("c")
    lo = wid * OWN
    hi = lo + OWN

    pltpu.sync_copy(ids_hbm, ids_v)
    pltpu.sync_copy(tail_hbm, tail_v)

    def _c0(t):
        return pl.multiple_of(jnp.minimum(lo + t * CHW, TAIL0 - CHW), 128)

    NCHUNKS = OWN // CHW
    slabs = [slab0, slab1]
    sems = [sem0, sem1]
    for b in range(2):
        pltpu.async_copy(tT_hbm.at[:, pl.ds(_c0(b), CHW)], slabs[b], sems[b])

    def _pf(i, c):
        oid[i, :] = _full16(SENT)
        return c
    lax.fori_loop(0, MAXO // 16 + 1, _pf, 0)

    def _scan(i, cnt):
        av = ids_v[pl.ds(i * 16, 16)]
        mi = _inrange(av, lo, hi)
        cs = plsc.cumsum(mi)
        pos = MAXO + mi * (cnt + cs - 1 - MAXO)
        ph = lax.shift_right_logical(pos, 4)
        plo = pos - ph * 16
        plsc.store_scatter(oid, [ph, plo], av)
        plsc.store_scatter(onn, [ph, plo], _iota16() + i * 16)
        return jnp.minimum(cnt + _splat_last(cs), MAXO - 16)
    lax.fori_loop(0, N // 16, _scan, _full16(0))

    dump = N + wid * MAXC
    def _emit_block(c0, c1, gather_fn):
        for k in range(MAXC // 16 + 1):
            cnn[k, :] = _iota16() + (dump + k * 16)
            cid[k, :] = _full16(0)

        def _filt(i, cc):
            av = oid[i, :]
            mi = _inrange(av, c0, c1)
            cs = plsc.cumsum(mi)
            pos = MAXC + mi * (cc + cs - 1 - MAXC)
            ph = lax.shift_right_logical(pos, 4)
            plo = pos - ph * 16
            plsc.store_scatter(cid, [ph, plo], (av - c0) * mi)
            plsc.store_scatter(cnn, [ph, plo], onn[i, :])
            return jnp.minimum(cc + _splat_last(cs), MAXC - 16)
        lax.fori_loop(0, MAXO // 16, _filt, _full16(0))

        def _grp(g, c):
            cols = cid[g, :]
            ridx = _iota16() + g * 16
            for d in range(D):
                plsc.store_scatter(stg, [ridx, _full16(d)],
                                   gather_fn(d, cols))
            return c
        lax.fori_loop(0, MAXC // 16, _grp, 0)

        for k in range(MAXC // 16):
            cnn2[0, pl.ds(k * 16, 16)] = cnn[k, :]
        pltpu.sync_copy(stg, out_hbm.at[cnn2.at[0]])

    def _ring(o, carry):
        for b in range(2):
            t = 2 * o + b
            pltpu.make_async_copy(tT_hbm.at[:, pl.ds(0, CHW)],
                                  slabs[b], sems[b]).wait()
            c0 = _c0(t)
            _emit_block(c0, c0 + CHW,
                        lambda d, cols, slab=slabs[b]:
                        plsc.load_gather(slab, [_full16(d), cols]))
            pltpu.async_copy(tT_hbm.at[:, pl.ds(_c0(t + 2), CHW)],
                             slabs[b], sems[b])
        return carry
    lax.fori_loop(0, (NCHUNKS - 1) // 2, _ring, 0)

    pltpu.make_async_copy(tT_hbm.at[:, pl.ds(0, CHW)],
                          slabs[0], sems[0]).wait()
    cL = _c0(NCHUNKS - 1)
    _emit_block(cL, cL + CHW,
                lambda d, cols: plsc.load_gather(slab0, [_full16(d), cols]))
    pltpu.make_async_copy(tT_hbm.at[:, pl.ds(0, CHW)],
                          slabs[1], sems[1]).wait()

    _emit_block(TAIL0, V,
                lambda d, cols: plsc.load_gather(tail_v, [cols, _full16(d)]))


def _ll_tc(rows_ref, mu_ref, anno_ref, conf_ref, out_ref):
    rows = rows_ref[:, :D]
    e_mu = jnp.exp(mu_ref[...])
    ee = jnp.exp(e_mu)
    er = jnp.exp(rows)
    s = lax.dot_general(er, ee, (((1,), (1,)), ((), ())),
                        preferred_element_type=jnp.float32)
    anno = anno_ref[...]
    onehot = (anno == lax.broadcasted_iota(jnp.int32, rows.shape, 1)
              ).astype(jnp.float32)
    r_an = jnp.sum(rows * onehot, axis=1, keepdims=True)
    e_an = lax.dot_general(onehot, e_mu, (((1,), (1,)), ((), ())),
                           preferred_element_type=jnp.float32)
    out_ref[...] = (e_an + r_an - jnp.log(s)) * conf_ref[...]


def _scatter_sc(ll_hbm, items_hbm, out_hbm, idx_v, ll_v, zbuf, acc_sh, sem):
    del sem
    cid_ = lax.axis_index("c")
    sid = lax.axis_index("s")
    wid = sid * NC + cid_
    base = wid * ROWS_PER_W
    pltpu.sync_copy(items_hbm.at[wid], idx_v)
    pltpu.sync_copy(ll_hbm.at[pl.ds(base, ROWS_PER_W)], ll_v)

    def _zero_row(j, carry):
        zbuf[j, :] = jnp.zeros((C,), jnp.float32)
        return carry
    lax.fori_loop(0, STRIPE, _zero_row, 0)
    pltpu.sync_copy(zbuf, acc_sh.at[pl.ds(sid * STRIPE, STRIPE)])
    plsc.subcore_barrier()
    for j in range(KCH):
        pltpu.sync_copy(ll_v.at[pl.ds(j * 128, 128)],
                        acc_sh.at[idx_v.at[j]], add=True)
    plsc.subcore_barrier()
    pltpu.sync_copy(acc_sh.at[pl.ds(sid * STRIPE, STRIPE)], zbuf)
    pltpu.sync_copy(zbuf, out_hbm.at[cid_, pl.ds(sid * STRIPE, STRIPE)])


def _combine_tc(parts_ref, out_ref):
    out_ref[...] = (parts_ref[0] + parts_ref[1]).T


def kernel(mu, random_effects, anno, items, annotators, confidences):
    mesh = plsc.VectorSubcoreMesh(core_axis_name="c", subcore_axis_name="s")

    gather = pl.kernel(
        _gather_sc, mesh=mesh,
        compiler_params=pltpu.CompilerParams(needs_layout_passes=False),
        out_type=jax.ShapeDtypeStruct((OUTROWS, 128), jnp.float32),
        scratch_types=[
            pltpu.VMEM((N,), jnp.int32),
            pltpu.VMEM((D, CHW), jnp.float32),
            pltpu.VMEM((D, CHW), jnp.float32),
            pltpu.VMEM((V - TAIL0, D), jnp.float32),
            pltpu.VMEM((MAXO // 16 + 1, 16), jnp.int32),
            pltpu.VMEM((MAXO // 16 + 1, 16), jnp.int32),
            pltpu.VMEM((MAXC // 16 + 1, 16), jnp.int32),
            pltpu.VMEM((MAXC // 16 + 1, 16), jnp.int32),
            pltpu.VMEM((1, MAXC), jnp.int32),
            pltpu.VMEM((MAXC, 128), jnp.float32),
            pltpu.SemaphoreType.DMA,
            pltpu.SemaphoreType.DMA,
        ],
    )
    rows4 = gather(random_effects.T, annotators.astype(jnp.int32),
                   lax.slice(random_effects, (TAIL0, 0), (V, D)))

    grid = 8
    blk = N // grid
    ll = pl.pallas_call(
        _ll_tc,
        grid=(grid,),
        in_specs=[
            pl.BlockSpec((blk, 128), lambda i: (i, 0)),
            pl.BlockSpec((C, D), lambda i: (0, 0)),
            pl.BlockSpec((blk, 1), lambda i: (i, 0)),
            pl.BlockSpec((blk, 1), lambda i: (i, 0)),
        ],
        out_specs=pl.BlockSpec((blk, C), lambda i: (i, 0)),
        out_shape=jax.ShapeDtypeStruct((N, C), jnp.float32),
    )(rows4, mu, anno.astype(jnp.int32).reshape(N, 1),
      confidences.reshape(N, 1))

    scatter = pl.kernel(
        _scatter_sc, mesh=mesh,
        compiler_params=pltpu.CompilerParams(use_tc_tiling_on_sc=False),
        out_type=jax.ShapeDtypeStruct((NC, I, C), jnp.float32),
        scratch_types=[
            pltpu.VMEM((KCH, 128), jnp.int32),
            pltpu.VMEM((ROWS_PER_W, C), jnp.float32),
            pltpu.VMEM((STRIPE, C), jnp.float32),
            pltpu.VMEM_SHARED((I, C), jnp.float32),
            pltpu.SemaphoreType.DMA,
        ],
    )
    parts = scatter(ll, items.astype(jnp.int32).reshape(NW, KCH, 128))

    return pl.pallas_call(
        _combine_tc,
        out_shape=jax.ShapeDtypeStruct((C, I), jnp.float32),
    )(parts)


# speedup vs baseline: 1.1512x; 1.0274x over previous
import jax
import jax.numpy as jnp
from jax import lax
from jax.experimental import pallas as pl
from jax.experimental.pallas import tpu as pltpu
from jax.experimental.pallas import tpu_sc as plsc

C = 16
D = 32
V = 1000000
N = 16384
I = 4096

NC = 2
NS = 16
NW = NC * NS
ROWS_PER_W = N // NW
KCH = ROWS_PER_W // 128
STRIPE = I // NS

OWN = 31744
CHW = 1024
TAIL0 = 999936
MAXO = 768
MAXC = 48
SENT = 1 << 29
OUTROWS = N + NW * MAXC


def _iota16():
    return lax.iota(jnp.int32, 16)


def _full16(v):
    return jnp.full((16,), v, jnp.int32)


def _splat_last(cs):
    dn = lax.GatherDimensionNumbers(offset_dims=(), collapsed_slice_dims=(0,),
                                    start_index_map=(0,))
    return lax.gather(cs, _full16(15).reshape(16, 1), dn, (1,),
                      mode=lax.GatherScatterMode.PROMISE_IN_BOUNDS)


def _inrange(av, c0, c1):
    t = lax.shift_right_logical(av - c0, 31)
    u = lax.shift_right_logical((c1 - 1) - av, 31)
    return (1 - t) * (1 - u)


def _gather_sc(tT_hbm, ids_hbm, tail_hbm, out_hbm,
               ids_v, slab0, slab1, tail_v, oid, onn, cid, cnn, cnn2, stg,
               sem0, sem1):
    wid = lax.axis_index("s") * NC + lax.axis_index("c")
    lo = wid * OWN
    hi = lo + OWN

    pltpu.sync_copy(ids_hbm, ids_v)
    pltpu.sync_copy(tail_hbm, tail_v)

    def _c0(t):
        return pl.multiple_of(jnp.minimum(lo + t * CHW, TAIL0 - CHW), 128)

    NCHUNKS = OWN // CHW
    slabs = [slab0, slab1]
    sems = [sem0, sem1]
    for b in range(2):
        pltpu.async_copy(tT_hbm.at[:, pl.ds(_c0(b), CHW)], slabs[b], sems[b])

    def _pf(i, c):
        oid[i, :] = _full16(SENT)
        return c
    lax.fori_loop(0, MAXO // 16 + 1, _pf, 0)

    def _scan(i, cnt):
        av = ids_v[pl.ds(i * 16, 16)]
        mi = _inrange(av, lo, hi)
        cs = plsc.cumsum(mi)
        pos = MAXO + mi * (cnt + cs - 1 - MAXO)
        ph = lax.shift_right_logical(pos, 4)
        plo = pos - ph * 16
        plsc.store_scatter(oid, [ph, plo], av)
        plsc.store_scatter(onn, [ph, plo], _iota16() + i * 16)
        return jnp.minimum(cnt + _splat_last(cs), MAXO - 16)
    cnt_fin = lax.fori_loop(0, N // 16, _scan, _full16(0))
    nvr = lax.shift_right_logical(jnp.max(cnt_fin) + 31, 4)

    dump = N + wid * MAXC
    def _emit_block(c0, c1, gather_fn):
        for k in range(MAXC // 16 + 1):
            cnn[k, :] = _iota16() + (dump + k * 16)
            cid[k, :] = _full16(0)

        def _filt(i, cc):
            av = oid[i, :]
            mi = _inrange(av, c0, c1)
            cs = plsc.cumsum(mi)
            pos = MAXC + mi * (cc + cs - 1 - MAXC)
            ph = lax.shift_right_logical(pos, 4)
            plo = pos - ph * 16
            plsc.store_scatter(cid, [ph, plo], (av - c0) * mi)
            plsc.store_scatter(cnn, [ph, plo], onn[i, :])
            return jnp.minimum(cc + _splat_last(cs), MAXC - 16)
        lax.fori_loop(0, nvr, _filt, _full16(0))

        def _grp(g, c):
            cols = cid[g, :]
            ridx = _iota16() + g * 16
            for d in range(D):
                plsc.store_scatter(stg, [ridx, _full16(d)],
                                   gather_fn(d, cols))
            return c
        lax.fori_loop(0, MAXC // 16, _grp, 0)

        for k in range(MAXC // 16):
            cnn2[0, pl.ds(k * 16, 16)] = cnn[k, :]
        pltpu.sync_copy(stg, out_hbm.at[cnn2.at[0]])

    def _ring(o, carry):
        for b in range(2):
            t = 2 * o + b
            pltpu.make_async_copy(tT_hbm.at[:, pl.ds(0, CHW)],
                                  slabs[b], sems[b]).wait()
            c0 = _c0(t)
            _emit_block(c0, c0 + CHW,
                        lambda d, cols, slab=slabs[b]:
                        plsc.load_gather(slab, [_full16(d), cols]))
            pltpu.async_copy(tT_hbm.at[:, pl.ds(_c0(t + 2), CHW)],
                             slabs[b], sems[b])
        return carry
    lax.fori_loop(0, (NCHUNKS - 1) // 2, _ring, 0)

    pltpu.make_async_copy(tT_hbm.at[:, pl.ds(0, CHW)],
                          slabs[0], sems[0]).wait()
    cL = _c0(NCHUNKS - 1)
    _emit_block(cL, cL + CHW,
                lambda d, cols: plsc.load_gather(slab0, [_full16(d), cols]))
    pltpu.make_async_copy(tT_hbm.at[:, pl.ds(0, CHW)],
                          slabs[1], sems[1]).wait()

    _emit_block(TAIL0, V,
                lambda d, cols: plsc.load_gather(tail_v, [cols, _full16(d)]))


def _ll_tc(rows_ref, mu_ref, anno_ref, conf_ref, out_ref):
    rows = rows_ref[:, :D]
    e_mu = jnp.exp(mu_ref[...])
    ee = jnp.exp(e_mu)
    er = jnp.exp(rows)
    s = lax.dot_general(er, ee, (((1,), (1,)), ((), ())),
                        preferred_element_type=jnp.float32)
    anno = anno_ref[...]
    onehot = (anno == lax.broadcasted_iota(jnp.int32, rows.shape, 1)
              ).astype(jnp.float32)
    r_an = jnp.sum(rows * onehot, axis=1, keepdims=True)
    e_an = lax.dot_general(onehot, e_mu, (((1,), (1,)), ((), ())),
                           preferred_element_type=jnp.float32)
    out_ref[...] = (e_an + r_an - jnp.log(s)) * conf_ref[...]


def _scatter_sc(ll_hbm, items_hbm, out_hbm, idx_v, ll_v, zbuf, acc_sh, sem):
    del sem
    cid_ = lax.axis_index("c")
    sid = lax.axis_index("s")
    wid = sid * NC + cid_
    base = wid * ROWS_PER_W
    pltpu.sync_copy(items_hbm.at[wid], idx_v)
    pltpu.sync_copy(ll_hbm.at[pl.ds(base, ROWS_PER_W)], ll_v)

    def _zero_row(j, carry):
        zbuf[j, :] = jnp.zeros((C,), jnp.float32)
        return carry
    lax.fori_loop(0, STRIPE, _zero_row, 0)
    pltpu.sync_copy(zbuf, acc_sh.at[pl.ds(sid * STRIPE, STRIPE)])
    plsc.subcore_barrier()
    for j in range(KCH):
        pltpu.sync_copy(ll_v.at[pl.ds(j * 128, 128)],
                        acc_sh.at[idx_v.at[j]], add=True)
    plsc.subcore_barrier()
    pltpu.sync_copy(acc_sh.at[pl.ds(sid * STRIPE, STRIPE)], zbuf)
    pltpu.sync_copy(zbuf, out_hbm.at[cid_, pl.ds(sid * STRIPE, STRIPE)])


def _combine_tc(parts_ref, out_ref):
    out_ref[...] = (parts_ref[0] + parts_ref[1]).T


def kernel(mu, random_effects, anno, items, annotators, confidences):
    mesh = plsc.VectorSubcoreMesh(core_axis_name="c", subcore_axis_name="s")

    gather = pl.kernel(
        _gather_sc, mesh=mesh,
        compiler_params=pltpu.CompilerParams(needs_layout_passes=False),
        out_type=jax.ShapeDtypeStruct((OUTROWS, 128), jnp.float32),
        scratch_types=[
            pltpu.VMEM((N,), jnp.int32),
            pltpu.VMEM((D, CHW), jnp.float32),
            pltpu.VMEM((D, CHW), jnp.float32),
            pltpu.VMEM((V - TAIL0, D), jnp.float32),
            pltpu.VMEM((MAXO // 16 + 1, 16), jnp.int32),
            pltpu.VMEM((MAXO // 16 + 1, 16), jnp.int32),
            pltpu.VMEM((MAXC // 16 + 1, 16), jnp.int32),
            pltpu.VMEM((MAXC // 16 + 1, 16), jnp.int32),
            pltpu.VMEM((1, MAXC), jnp.int32),
            pltpu.VMEM((MAXC, 128), jnp.float32),
            pltpu.SemaphoreType.DMA,
            pltpu.SemaphoreType.DMA,
        ],
    )
    rows4 = gather(random_effects.T, annotators.astype(jnp.int32),
                   lax.slice(random_effects, (TAIL0, 0), (V, D)))

    grid = 8
    blk = N // grid
    ll = pl.pallas_call(
        _ll_tc,
        grid=(grid,),
        in_specs=[
            pl.BlockSpec((blk, 128), lambda i: (i, 0)),
            pl.BlockSpec((C, D), lambda i: (0, 0)),
            pl.BlockSpec((blk, 1), lambda i: (i, 0)),
            pl.BlockSpec((blk, 1), lambda i: (i, 0)),
        ],
        out_specs=pl.BlockSpec((blk, C), lambda i: (i, 0)),
        out_shape=jax.ShapeDtypeStruct((N, C), jnp.float32),
    )(rows4, mu, anno.astype(jnp.int32).reshape(N, 1),
      confidences.reshape(N, 1))

    scatter = pl.kernel(
        _scatter_sc, mesh=mesh,
        compiler_params=pltpu.CompilerParams(use_tc_tiling_on_sc=False),
        out_type=jax.ShapeDtypeStruct((NC, I, C), jnp.float32),
        scratch_types=[
            pltpu.VMEM((KCH, 128), jnp.int32),
            pltpu.VMEM((ROWS_PER_W, C), jnp.float32),
            pltpu.VMEM((STRIPE, C), jnp.float32),
            pltpu.VMEM_SHARED((I, C), jnp.float32),
            pltpu.SemaphoreType.DMA,
        ],
    )
    parts = scatter(ll, items.astype(jnp.int32).reshape(NW, KCH, 128))

    return pl.pallas_call(
        _combine_tc,
        out_shape=jax.ShapeDtypeStruct((C, I), jnp.float32),
    )(parts)


# speedup vs baseline: 1.1531x; 1.0017x over previous
import jax
import jax.numpy as jnp
from jax import lax
from jax.experimental import pallas as pl
from jax.experimental.pallas import tpu as pltpu
from jax.experimental.pallas import tpu_sc as plsc

C = 16
D = 32
V = 1000000
N = 16384
I = 4096

NC = 2
NS = 16
NW = NC * NS
ROWS_PER_W = N // NW
KCH = ROWS_PER_W // 128
STRIPE = I // NS

OWN = 31744
CHW = 1024
TAIL0 = 999936
MAXO = 768
MAXC = 48
SENT = 1 << 29
OUTROWS = N + NW * MAXC


def _iota16():
    return lax.iota(jnp.int32, 16)


def _full16(v):
    return jnp.full((16,), v, jnp.int32)


def _splat_last(cs):
    dn = lax.GatherDimensionNumbers(offset_dims=(), collapsed_slice_dims=(0,),
                                    start_index_map=(0,))
    return lax.gather(cs, _full16(15).reshape(16, 1), dn, (1,),
                      mode=lax.GatherScatterMode.PROMISE_IN_BOUNDS)


def _inrange(av, c0, c1):
    t = lax.shift_right_logical(av - c0, 31)
    u = lax.shift_right_logical((c1 - 1) - av, 31)
    return (1 - t) * (1 - u)


def _gather_sc(tT_hbm, ids_hbm, tail_hbm, out_hbm,
               ids_v, slab0, slab1, tail_v, oid, onn, cid, cnn, cnn2, stg,
               sem0, sem1):
    wid = lax.axis_index("s") * NC + lax.axis_index("c")
    lo = wid * OWN
    hi = lo + OWN

    pltpu.sync_copy(ids_hbm, ids_v)
    pltpu.sync_copy(tail_hbm, tail_v)

    def _c0(t):
        return pl.multiple_of(jnp.minimum(lo + t * CHW, TAIL0 - CHW), 128)

    NCHUNKS = OWN // CHW
    slabs = [slab0, slab1]
    sems = [sem0, sem1]
    for b in range(2):
        pltpu.async_copy(tT_hbm.at[:, pl.ds(_c0(b), CHW)], slabs[b], sems[b])

    def _pf(i, c):
        oid[i, :] = _full16(SENT)
        return c
    lax.fori_loop(0, MAXO // 16 + 1, _pf, 0)

    def _scan(i, cnt):
        av = ids_v[pl.ds(i * 16, 16)]
        mi = _inrange(av, lo, hi)
        cs = plsc.cumsum(mi)
        pos = MAXO + mi * (cnt + cs - 1 - MAXO)
        ph = lax.shift_right_logical(pos, 4)
        plo = pos - ph * 16
        plsc.store_scatter(oid, [ph, plo], av)
        plsc.store_scatter(onn, [ph, plo], _iota16() + i * 16)
        return jnp.minimum(cnt + _splat_last(cs), MAXO - 16)
    cnt_fin = lax.fori_loop(0, N // 16, _scan, _full16(0))
    nvr = lax.shift_right_logical(jnp.max(cnt_fin) + 31, 4)

    dump = N + wid * MAXC
    def _emit_block(c0, c1, gather_fn):
        for k in range(MAXC // 16 + 1):
            cnn[k, :] = _iota16() + (dump + k * 16)
            cid[k, :] = _full16(0)

        def _filt(i, cc):
            av = oid[i, :]
            mi = _inrange(av, c0, c1)
            cs = plsc.cumsum(mi)
            pos = MAXC + mi * (cc + cs - 1 - MAXC)
            ph = lax.shift_right_logical(pos, 4)
            plo = pos - ph * 16
            plsc.store_scatter(cid, [ph, plo], (av - c0) * mi)
            plsc.store_scatter(cnn, [ph, plo], onn[i, :])
            return jnp.minimum(cc + _splat_last(cs), MAXC - 16)
        lax.fori_loop(0, nvr, _filt, _full16(0))

        def _grp(g, c):
            cols = cid[g, :]
            ridx = _iota16() + g * 16
            for d in range(D):
                plsc.store_scatter(stg, [ridx, _full16(d)],
                                   gather_fn(d, cols))
            return c
        lax.fori_loop(0, MAXC // 16, _grp, 0)

        for k in range(MAXC // 16):
            cnn2[0, pl.ds(k * 16, 16)] = cnn[k, :]
        pltpu.sync_copy(stg, out_hbm.at[cnn2.at[0]])

    def _ring(o, carry):
        for b in range(2):
            t = 2 * o + b
            pltpu.make_async_copy(tT_hbm.at[:, pl.ds(0, CHW)],
                                  slabs[b], sems[b]).wait()
            c0 = _c0(t)
            _emit_block(c0, c0 + CHW,
                        lambda d, cols, slab=slabs[b]:
                        plsc.load_gather(slab, [_full16(d), cols]))
            pltpu.async_copy(tT_hbm.at[:, pl.ds(_c0(t + 2), CHW)],
                             slabs[b], sems[b])
        return carry
    lax.fori_loop(0, (NCHUNKS - 1) // 2, _ring, 0)

    pltpu.make_async_copy(tT_hbm.at[:, pl.ds(0, CHW)],
                          slabs[0], sems[0]).wait()
    cL = _c0(NCHUNKS - 1)
    _emit_block(cL, cL + CHW,
                lambda d, cols: plsc.load_gather(slab0, [_full16(d), cols]))
    pltpu.make_async_copy(tT_hbm.at[:, pl.ds(0, CHW)],
                          slabs[1], sems[1]).wait()

    def _tail_gather(d, cols):
        fl = cols * D + d
        fh = lax.shift_right_logical(fl, 7)
        return plsc.load_gather(tail_v, [fh, fl - fh * 128])
    _emit_block(TAIL0, V, _tail_gather)


def _ll_tc(rows_ref, mu_ref, anno_ref, conf_ref, out_ref):
    rows = rows_ref[:, :D]
    e_mu = jnp.exp(mu_ref[...])
    ee = jnp.exp(e_mu)
    er = jnp.exp(rows)
    s = lax.dot_general(er, ee, (((1,), (1,)), ((), ())),
                        preferred_element_type=jnp.float32)
    anno = anno_ref[...]
    onehot = (anno == lax.broadcasted_iota(jnp.int32, rows.shape, 1)
              ).astype(jnp.float32)
    r_an = jnp.sum(rows * onehot, axis=1, keepdims=True)
    e_an = lax.dot_general(onehot, e_mu, (((1,), (1,)), ((), ())),
                           preferred_element_type=jnp.float32)
    out_ref[...] = (e_an + r_an - jnp.log(s)) * conf_ref[...]


def _scatter_sc(ll_hbm, items_hbm, out_hbm, idx_v, ll_v, zbuf, acc_sh, sem):
    del sem
    cid_ = lax.axis_index("c")
    sid = lax.axis_index("s")
    wid = sid * NC + cid_
    base = wid * ROWS_PER_W
    pltpu.sync_copy(items_hbm.at[wid], idx_v)
    pltpu.sync_copy(ll_hbm.at[pl.ds(base, ROWS_PER_W)], ll_v)

    def _zero_row(j, carry):
        zbuf[j, :] = jnp.zeros((C,), jnp.float32)
        return carry
    lax.fori_loop(0, STRIPE, _zero_row, 0)
    pltpu.sync_copy(zbuf, acc_sh.at[pl.ds(sid * STRIPE, STRIPE)])
    plsc.subcore_barrier()
    for j in range(KCH):
        pltpu.sync_copy(ll_v.at[pl.ds(j * 128, 128)],
                        acc_sh.at[idx_v.at[j]], add=True)
    plsc.subcore_barrier()
    pltpu.sync_copy(acc_sh.at[pl.ds(sid * STRIPE, STRIPE)], zbuf)
    pltpu.sync_copy(zbuf, out_hbm.at[cid_, pl.ds(sid * STRIPE, STRIPE)])


def _combine_tc(parts_ref, out_ref):
    out_ref[...] = (parts_ref[0] + parts_ref[1]).T


def kernel(mu, random_effects, anno, items, annotators, confidences):
    mesh = plsc.VectorSubcoreMesh(core_axis_name="c", subcore_axis_name="s")

    gather = pl.kernel(
        _gather_sc, mesh=mesh,
        compiler_params=pltpu.CompilerParams(needs_layout_passes=False),
        out_type=jax.ShapeDtypeStruct((OUTROWS, 128), jnp.float32),
        scratch_types=[
            pltpu.VMEM((N,), jnp.int32),
            pltpu.VMEM((D, CHW), jnp.float32),
            pltpu.VMEM((D, CHW), jnp.float32),
            pltpu.VMEM(((V - TAIL0) * D // 128, 128), jnp.float32),
            pltpu.VMEM((MAXO // 16 + 1, 16), jnp.int32),
            pltpu.VMEM((MAXO // 16 + 1, 16), jnp.int32),
            pltpu.VMEM((MAXC // 16 + 1, 16), jnp.int32),
            pltpu.VMEM((MAXC // 16 + 1, 16), jnp.int32),
            pltpu.VMEM((1, MAXC), jnp.int32),
            pltpu.VMEM((MAXC, 128), jnp.float32),
            pltpu.SemaphoreType.DMA,
            pltpu.SemaphoreType.DMA,
        ],
    )
    rows4 = gather(random_effects.T, annotators.astype(jnp.int32),
                   lax.slice(random_effects, (TAIL0, 0), (V, D))
                   .reshape((V - TAIL0) * D // 128, 128))

    grid = 8
    blk = N // grid
    ll = pl.pallas_call(
        _ll_tc,
        grid=(grid,),
        in_specs=[
            pl.BlockSpec((blk, 128), lambda i: (i, 0)),
            pl.BlockSpec((C, D), lambda i: (0, 0)),
            pl.BlockSpec((blk, 1), lambda i: (i, 0)),
            pl.BlockSpec((blk, 1), lambda i: (i, 0)),
        ],
        out_specs=pl.BlockSpec((blk, C), lambda i: (i, 0)),
        out_shape=jax.ShapeDtypeStruct((N, C), jnp.float32),
    )(rows4, mu, anno.astype(jnp.int32).reshape(N, 1),
      confidences.reshape(N, 1))

    scatter = pl.kernel(
        _scatter_sc, mesh=mesh,
        compiler_params=pltpu.CompilerParams(use_tc_tiling_on_sc=False),
        out_type=jax.ShapeDtypeStruct((NC, I, C), jnp.float32),
        scratch_types=[
            pltpu.VMEM((KCH, 128), jnp.int32),
            pltpu.VMEM((ROWS_PER_W, C), jnp.float32),
            pltpu.VMEM((STRIPE, C), jnp.float32),
            pltpu.VMEM_SHARED((I, C), jnp.float32),
            pltpu.SemaphoreType.DMA,
        ],
    )
    parts = scatter(ll, items.astype(jnp.int32).reshape(NW, KCH, 128))

    return pl.pallas_call(
        _combine_tc,
        out_shape=jax.ShapeDtypeStruct((C, I), jnp.float32),
    )(parts)
